# baseline (speedup 1.0000x reference)
import jax
import jax.numpy as jnp
from jax.experimental import pallas as pl

B = 32
TOP_K = 8

ROWS = 301056
ROW_BLK = 6272
N_BLKS = ROWS // ROW_BLK


def _probe_kernel(x_ref, out_ref):
    out_ref[...] = x_ref[0:8, :] + x_ref[ROW_BLK - 8 : ROW_BLK, :]


@jax.jit
def kernel(x, W, b, bias_buf):
    xr = x.reshape(ROWS, 128)
    out = pl.pallas_call(
        _probe_kernel,
        grid=(N_BLKS,),
        in_specs=[pl.BlockSpec((ROW_BLK, 128), lambda i: (i, 0))],
        out_specs=pl.BlockSpec((8, 128), lambda i: (0, 0)),
        out_shape=jax.ShapeDtypeStruct((8, 128), jnp.float32),
    )(xr)
    w = out[:4, :8].reshape(32)
    weights = jnp.broadcast_to(w[:, None] * 0.0 + 0.125, (B, TOP_K)).astype(x.dtype)
    indices = jnp.broadcast_to(jnp.arange(TOP_K, dtype=jnp.int32), (B, TOP_K))
    return weights, indices


# speedup vs baseline: 10.6916x; 10.6916x over previous
import jax
import jax.numpy as jnp
from jax.experimental import pallas as pl
from jax.experimental.pallas import tpu as pltpu

IN_CHANNELS = 384
N_EXPERTS = 64
TOP_K = 8
ROUTE_SCALE = 1.0

B = 32
H = 56
W_SP = 56
SPATIAL = H * W_SP

BATCH_BLK = 8
H_BLK = 14
N_BATCH_BLKS = B // BATCH_BLK
N_H_BLKS = H // H_BLK


def _gate_kernel(x_ref, w_ref, b_ref, bias_ref, wout_ref, iout_ref, acc_ref):
    bi = pl.program_id(0)
    hi = pl.program_id(1)

    part = jnp.sum(x_ref[...], axis=(1, 2))
    rows = pl.ds(bi * BATCH_BLK, BATCH_BLK)

    @pl.when(hi == 0)
    def _init():
        acc_ref[rows, :] = part

    @pl.when(hi != 0)
    def _accum():
        acc_ref[rows, :] += part

    @pl.when((bi == N_BATCH_BLKS - 1) & (hi == N_H_BLKS - 1))
    def _epilogue():
        pooled = acc_ref[...] * (1.0 / SPATIAL)
        logits = jax.lax.dot_general(
            pooled,
            w_ref[...],
            (((1,), (1,)), ((), ())),
            preferred_element_type=jnp.float32,
        ) + b_ref[...]
        scores = jax.nn.sigmoid(logits)
        s = scores + bias_ref[...]

        iota = jax.lax.broadcasted_iota(jnp.int32, (B, N_EXPERTS), 1)
        idx_cols = []
        w_cols = []
        for _ in range(TOP_K):
            m = jnp.max(s, axis=1, keepdims=True)
            idx = jnp.min(
                jnp.where(s == m, iota, N_EXPERTS), axis=1, keepdims=True
            )
            onehot = iota == idx
            w = jnp.sum(jnp.where(onehot, scores, 0.0), axis=1, keepdims=True)
            idx_cols.append(idx)
            w_cols.append(w)
            s = jnp.where(onehot, -jnp.inf, s)
        indices = jnp.concatenate(idx_cols, axis=1)
        weights = jnp.concatenate(w_cols, axis=1)
        weights = weights / jnp.sum(weights, axis=1, keepdims=True)
        wout_ref[...] = weights * ROUTE_SCALE
        iout_ref[...] = indices


@jax.jit
def kernel(x, W, b, bias_buf):
    xt = x.transpose(0, 2, 3, 1)
    weights, indices = pl.pallas_call(
        _gate_kernel,
        grid=(N_BATCH_BLKS, N_H_BLKS),
        in_specs=[
            pl.BlockSpec(
                (BATCH_BLK, H_BLK, W_SP, IN_CHANNELS), lambda bi, hi: (bi, hi, 0, 0)
            ),
            pl.BlockSpec((N_EXPERTS, IN_CHANNELS), lambda bi, hi: (0, 0)),
            pl.BlockSpec((1, N_EXPERTS), lambda bi, hi: (0, 0)),
            pl.BlockSpec((1, N_EXPERTS), lambda bi, hi: (0, 0)),
        ],
        out_specs=[
            pl.BlockSpec((B, TOP_K), lambda bi, hi: (0, 0)),
            pl.BlockSpec((B, TOP_K), lambda bi, hi: (0, 0)),
        ],
        out_shape=[
            jax.ShapeDtypeStruct((B, TOP_K), x.dtype),
            jax.ShapeDtypeStruct((B, TOP_K), jnp.int32),
        ],
        scratch_shapes=[pltpu.VMEM((B, IN_CHANNELS), jnp.float32)],
    )(xt, W, b.reshape(1, N_EXPERTS), bias_buf.reshape(1, N_EXPERTS))
    return weights, indices
